# baseline (speedup 1.0000x reference)
import functools

import jax
import jax.numpy as jnp
from jax.experimental import pallas as pl
from jax.experimental.pallas import tpu as pltpu


def _center_loss_kernel(feat_ref, label_ref, centers_ref, out_ref, *, inv_scale):
    i = pl.program_id(0)
    f = feat_ref[...]
    lab = label_ref[...]
    c = centers_ref[...]
    f2 = jnp.sum(f * f, axis=1, keepdims=True)
    c2 = jnp.sum(c * c, axis=1)[None, :]
    cross = jax.lax.dot_general(
        f.astype(jnp.bfloat16), c.astype(jnp.bfloat16),
        (((1,), (1,)), ((), ())),
        preferred_element_type=jnp.float32)
    partial = jnp.sum(lab * (f2 + c2 - 2.0 * cross))

    @pl.when(i == 0)
    def _():
        out_ref[0, 0] = 0.0

    out_ref[0, 0] += partial * inv_scale


def kernel(feat, label, centers):
    B, D = feat.shape
    C = label.shape[1]
    TB = 512 if B % 512 == 0 else B
    out = pl.pallas_call(
        functools.partial(_center_loss_kernel, inv_scale=1.0 / (2.0 * B * C)),
        grid=(B // TB,),
        in_specs=[
            pl.BlockSpec((TB, D), lambda i: (i, 0)),
            pl.BlockSpec((TB, C), lambda i: (i, 0)),
            pl.BlockSpec((C, D), lambda i: (0, 0)),
        ],
        out_specs=pl.BlockSpec((1, 1), lambda i: (0, 0), memory_space=pltpu.SMEM),
        out_shape=jax.ShapeDtypeStruct((1, 1), jnp.float32),
    )(feat, label, centers)
    return out[0, 0]


# speedup vs baseline: 1.0523x; 1.0523x over previous
import functools

import jax
import jax.numpy as jnp
from jax.experimental import pallas as pl
from jax.experimental.pallas import tpu as pltpu


def _center_loss_kernel(feat_ref, label_ref, centers_ref, out_ref,
                        acc_ref, acc2_ref, *, inv_scale, nsteps):
    i = pl.program_id(0)
    f = feat_ref[...]
    lab = label_ref[...].astype(jnp.bfloat16)
    fb = f.astype(jnp.bfloat16)
    f2 = jnp.sum(f * f, axis=1, keepdims=True)
    h = jnp.concatenate([f2, jnp.ones_like(f2)], axis=1).astype(jnp.bfloat16)

    m = jax.lax.dot_general(lab, fb, (((0,), (0,)), ((), ())),
                            preferred_element_type=jnp.float32)
    m2 = jax.lax.dot_general(lab, h, (((0,), (0,)), ((), ())),
                             preferred_element_type=jnp.float32)

    @pl.when(i == 0)
    def _():
        acc_ref[...] = m
        acc2_ref[...] = m2

    @pl.when(i > 0)
    def _():
        acc_ref[...] += m
        acc2_ref[...] += m2

    @pl.when(i == nsteps - 1)
    def _():
        c = centers_ref[...]
        acc = acc_ref[...]
        acc2 = acc2_ref[...]
        c2 = jnp.sum(c * c, axis=1)
        term12 = jnp.sum(acc2[:, 0]) + jnp.sum(c2 * acc2[:, 1])
        term3 = jnp.sum(c * acc)
        out_ref[0, 0] = (term12 - 2.0 * term3) * inv_scale


def kernel(feat, label, centers):
    B, D = feat.shape
    C = label.shape[1]
    TB = 1024 if B % 1024 == 0 else B
    nsteps = B // TB
    out = pl.pallas_call(
        functools.partial(_center_loss_kernel,
                          inv_scale=1.0 / (2.0 * B * C), nsteps=nsteps),
        grid=(nsteps,),
        in_specs=[
            pl.BlockSpec((TB, D), lambda i: (i, 0)),
            pl.BlockSpec((TB, C), lambda i: (i, 0)),
            pl.BlockSpec((C, D), lambda i: (0, 0)),
        ],
        out_specs=pl.BlockSpec((1, 1), lambda i: (0, 0), memory_space=pltpu.SMEM),
        out_shape=jax.ShapeDtypeStruct((1, 1), jnp.float32),
        scratch_shapes=[
            pltpu.VMEM((C, D), jnp.float32),
            pltpu.VMEM((C, 2), jnp.float32),
        ],
    )(feat, label, centers)
    return out[0, 0]


# speedup vs baseline: 2.2822x; 2.1687x over previous
import functools

import jax
import jax.numpy as jnp
from jax.experimental import pallas as pl
from jax.experimental.pallas import tpu as pltpu


def _center_loss_kernel(feat_ref, lt_ref, centers_ref, out_ref,
                        acc_ref, acc2_ref, *, inv_scale, nsteps):
    i = pl.program_id(0)
    f = feat_ref[...]
    lab = lt_ref[...].astype(jnp.bfloat16)
    fb = f.astype(jnp.bfloat16)
    f2 = jnp.sum(f * f, axis=1, keepdims=True)
    h = jnp.concatenate([f2, jnp.ones_like(f2)], axis=1).astype(jnp.bfloat16)

    m = jax.lax.dot_general(lab, fb, (((1,), (0,)), ((), ())),
                            preferred_element_type=jnp.float32)
    m2 = jax.lax.dot_general(lab, h, (((1,), (0,)), ((), ())),
                             preferred_element_type=jnp.float32)

    @pl.when(i == 0)
    def _():
        acc_ref[...] = m
        acc2_ref[...] = m2

    @pl.when(i > 0)
    def _():
        acc_ref[...] += m
        acc2_ref[...] += m2

    @pl.when(i == nsteps - 1)
    def _():
        c = centers_ref[...]
        acc = acc_ref[...]
        acc2 = acc2_ref[...]
        c2 = jnp.sum(c * c, axis=1)
        term12 = jnp.sum(acc2[:, 0]) + jnp.sum(c2 * acc2[:, 1])
        term3 = jnp.sum(c * acc)
        out_ref[0, 0] = (term12 - 2.0 * term3) * inv_scale


def kernel(feat, label, centers):
    B, D = feat.shape
    C = label.shape[1]
    lt = label.T
    TB = 1024 if B % 1024 == 0 else B
    nsteps = B // TB
    out = pl.pallas_call(
        functools.partial(_center_loss_kernel,
                          inv_scale=1.0 / (2.0 * B * C), nsteps=nsteps),
        grid=(nsteps,),
        in_specs=[
            pl.BlockSpec((TB, D), lambda i: (i, 0)),
            pl.BlockSpec((C, TB), lambda i: (0, i)),
            pl.BlockSpec((C, D), lambda i: (0, 0)),
        ],
        out_specs=pl.BlockSpec((1, 1), lambda i: (0, 0), memory_space=pltpu.SMEM),
        out_shape=jax.ShapeDtypeStruct((1, 1), jnp.float32),
        scratch_shapes=[
            pltpu.VMEM((C, D), jnp.float32),
            pltpu.VMEM((C, 2), jnp.float32),
        ],
    )(feat, lt, centers)
    return out[0, 0]


# speedup vs baseline: 2.2850x; 1.0012x over previous
import functools

import jax
import jax.numpy as jnp
from jax.experimental import pallas as pl
from jax.experimental.pallas import tpu as pltpu


def _center_loss_kernel(feat_ref, lt_ref, centers_ref, out_ref,
                        acc_ref, *, inv_scale, nsteps, ncols):
    i = pl.program_id(0)
    f = feat_ref[...]
    lab = lt_ref[...].astype(jnp.bfloat16)
    fb = f.astype(jnp.bfloat16)
    f2 = jnp.sum(f * f, axis=1, keepdims=True)
    g = jnp.concatenate(
        [fb, f2.astype(jnp.bfloat16), jnp.ones_like(fb[:, :1])], axis=1)

    m = jax.lax.dot_general(lab, g, (((1,), (0,)), ((), ())),
                            preferred_element_type=jnp.float32)

    @pl.when(i == 0)
    def _():
        acc_ref[...] = m

    @pl.when(i > 0)
    def _():
        acc_ref[...] += m

    @pl.when(i == nsteps - 1)
    def _():
        c = centers_ref[...]
        acc = acc_ref[...]
        c2 = jnp.sum(c * c, axis=1)
        term12 = jnp.sum(acc[:, ncols - 2]) + jnp.sum(c2 * acc[:, ncols - 1])
        term3 = jnp.sum(c * acc[:, :ncols - 2])
        out_ref[0, 0] = (term12 - 2.0 * term3) * inv_scale


def kernel(feat, label, centers):
    B, D = feat.shape
    C = label.shape[1]
    lt = label.T
    TB = 1024 if B % 1024 == 0 else B
    nsteps = B // TB
    ncols = D + 2
    out = pl.pallas_call(
        functools.partial(_center_loss_kernel,
                          inv_scale=1.0 / (2.0 * B * C),
                          nsteps=nsteps, ncols=ncols),
        grid=(nsteps,),
        in_specs=[
            pl.BlockSpec((TB, D), lambda i: (i, 0)),
            pl.BlockSpec((C, TB), lambda i: (0, i)),
            pl.BlockSpec((C, D), lambda i: (0, 0)),
        ],
        out_specs=pl.BlockSpec((1, 1), lambda i: (0, 0), memory_space=pltpu.SMEM),
        out_shape=jax.ShapeDtypeStruct((1, 1), jnp.float32),
        scratch_shapes=[
            pltpu.VMEM((C, ncols), jnp.float32),
        ],
    )(feat, lt, centers)
    return out[0, 0]
